# baseline (speedup 1.0000x reference)
import math

import jax
import jax.numpy as jnp
from jax import lax
from jax.experimental import pallas as pl
from jax.experimental.pallas import tpu as pltpu
from jax.experimental.pallas import tpu_sc as plsc

D_MODEL = 1024
D_FF = 4096
N_EXP = 8
TOK = 4096
BT = 256
NT = TOK // BT

G = 128
PADTOT = 2 * TOK + N_EXP * G
NTILES = PADTOT // G

NW = 32
TPW = TOK // NW
CHUNK = 32

_SQRT_HALF = 1.0 / math.sqrt(2.0)


def _gelu_exact(h):
    return 0.5 * h * (1.0 + jax.lax.erf(h * _SQRT_HALF))


def _first_occurrence_top2(s):
    jj = jax.lax.broadcasted_iota(jnp.int32, (N_EXP, N_EXP), 0)
    kk = jax.lax.broadcasted_iota(jnp.int32, (N_EXP, N_EXP), 1)
    tri = (jj < kk).astype(jnp.float32)
    neg = jnp.float32(-jnp.inf)

    m1 = jnp.max(s, axis=-1, keepdims=True)
    eq1 = (s == m1).astype(jnp.float32)
    prior1 = jnp.dot(eq1, tri, preferred_element_type=jnp.float32)
    first1 = eq1 * (prior1 == 0.0).astype(jnp.float32)

    s2 = jnp.where(first1 > 0.0, neg, s)
    m2 = jnp.max(s2, axis=-1, keepdims=True)
    eq2 = (s2 == m2).astype(jnp.float32)
    prior2 = jnp.dot(eq2, tri, preferred_element_type=jnp.float32)
    first2 = eq2 * (prior2 == 0.0).astype(jnp.float32)
    return first1, first2


def _gate_kernel(x_ref, wg_ref, bg_ref, mi_ref, mf_ref, ms_ref, carry_ref):
    t = pl.program_id(0)
    xb = x_ref[...]
    logits = jnp.dot(xb, wg_ref[...], preferred_element_type=jnp.float32)
    logits = logits + bg_ref[...]
    s = jax.nn.softmax(logits, axis=-1)
    first1, first2 = _first_occurrence_top2(s)

    lane = jax.lax.broadcasted_iota(jnp.int32, (BT, N_EXP), 1)
    lanef = lane.astype(jnp.float32)
    w1 = jnp.sum(s * first1, axis=-1, keepdims=True)
    w2 = jnp.sum(s * first2, axis=-1, keepdims=True)
    e1 = jnp.sum(lanef * first1, axis=-1, keepdims=True)
    e2 = jnp.sum(lanef * first2, axis=-1, keepdims=True)
    m = first1 + first2

    @pl.when(t == 0)
    def _():
        carry_ref[...] = jnp.zeros((1, N_EXP), jnp.float32)

    ii = jax.lax.broadcasted_iota(jnp.int32, (BT, BT), 0)
    jjb = jax.lax.broadcasted_iota(jnp.int32, (BT, BT), 1)
    trilS = (jjb < ii).astype(jnp.float32)
    carry = carry_ref[...]
    r = jnp.dot(trilS, m, preferred_element_type=jnp.float32) + carry
    carry_ref[...] = carry + jnp.sum(m, axis=0, keepdims=True)
    r1 = jnp.sum(r * first1, axis=-1, keepdims=True)
    r2 = jnp.sum(r * first2, axis=-1, keepdims=True)

    mf_ref[...] = jnp.where(lane == 0, w1, 0.0) + jnp.where(lane == 1, w2, 0.0)
    metai = (jnp.where(lane == 0, e1, 0.0) + jnp.where(lane == 1, e2, 0.0)
             + jnp.where(lane == 2, r1, 0.0) + jnp.where(lane == 3, r2, 0.0))

    ident = (ii == jjb).astype(jnp.float32)
    mi_t = lax.dot_general(metai, ident, (((0,), (0,)), ((), ())),
                           preferred_element_type=jnp.float32,
                           precision=lax.Precision.HIGHEST)
    mi_ref[...] = mi_t.astype(jnp.int32)

    @pl.when(t == NT - 1)
    def _():
        c = carry_ref[...]
        pc = jnp.floor((c + (G - 1)) / G) * G
        j8 = jax.lax.broadcasted_iota(jnp.int32, (N_EXP, N_EXP), 0)
        k8 = jax.lax.broadcasted_iota(jnp.int32, (N_EXP, N_EXP), 1)
        tri8 = (j8 < k8).astype(jnp.float32)
        off = jnp.dot(pc, tri8, preferred_element_type=jnp.float32)
        ends = off + pc
        j16 = jax.lax.broadcasted_iota(jnp.int32, (N_EXP, 16), 0)
        k16 = jax.lax.broadcasted_iota(jnp.int32, (N_EXP, 16), 1)
        p1 = (k16 == j16).astype(jnp.float32)
        p2 = (k16 == j16 + N_EXP).astype(jnp.float32)
        ms_ref[...] = (jnp.dot(off, p1, preferred_element_type=jnp.float32)
                       + jnp.dot(ends, p2, preferred_element_type=jnp.float32)
                       ).astype(jnp.int32)


@jax.jit
def _gate(x2d, Wg, bg2d):
    return pl.pallas_call(
        _gate_kernel,
        grid=(NT,),
        in_specs=[
            pl.BlockSpec((BT, D_MODEL), lambda t: (t, 0)),
            pl.BlockSpec((D_MODEL, N_EXP), lambda t: (0, 0)),
            pl.BlockSpec((1, N_EXP), lambda t: (0, 0)),
        ],
        out_specs=[
            pl.BlockSpec((N_EXP, BT), lambda t: (0, t)),
            pl.BlockSpec((BT, N_EXP), lambda t: (t, 0)),
            pl.BlockSpec((1, 16), lambda t: (0, 0)),
        ],
        out_shape=[
            jax.ShapeDtypeStruct((N_EXP, TOK), jnp.int32),
            jax.ShapeDtypeStruct((TOK, N_EXP), jnp.float32),
            jax.ShapeDtypeStruct((1, 16), jnp.int32),
        ],
        scratch_shapes=[pltpu.VMEM((1, N_EXP), jnp.float32)],
    )(x2d, Wg, bg2d)


def _pos_kernel(mi_ref, ms_ref, p_ref):
    e1 = mi_ref[0:1, :].astype(jnp.float32)
    e2 = mi_ref[1:2, :].astype(jnp.float32)
    r1 = mi_ref[2:3, :].astype(jnp.float32)
    r2 = mi_ref[3:4, :].astype(jnp.float32)
    offv = ms_ref[...].astype(jnp.float32)

    kk = jax.lax.broadcasted_iota(jnp.int32, (16, BT), 0).astype(jnp.float32)
    oh1 = (e1 == kk).astype(jnp.float32)
    oh2 = (e2 == kk).astype(jnp.float32)
    off1 = jnp.dot(offv, oh1, preferred_element_type=jnp.float32)
    off2 = jnp.dot(offv, oh2, preferred_element_type=jnp.float32)
    p1 = (r1 + off1).astype(jnp.int32)
    p2 = (r2 + off2).astype(jnp.int32)
    p_ref[...] = jnp.concatenate([p1, p2], axis=0)


@jax.jit
def _positions(mi, ms):
    return pl.pallas_call(
        _pos_kernel,
        grid=(NT,),
        in_specs=[
            pl.BlockSpec((N_EXP, BT), lambda t: (0, t)),
            pl.BlockSpec((1, 16), lambda t: (0, 0)),
        ],
        out_specs=pl.BlockSpec((2, BT), lambda t: (0, t)),
        out_shape=jax.ShapeDtypeStruct((2, TOK), jnp.int32),
    )(mi, ms)


def _scatter_body(x_hbm, p1_hbm, p2_hbm, xs_hbm, p1_v, p2_v, xrows_v, sem):
    wid = lax.axis_index("s") * 2 + lax.axis_index("c")
    base = wid * TPW
    for c in range(TPW // CHUNK):
        cb = base + c * CHUNK
        pltpu.sync_copy(p1_hbm.at[pl.ds(cb, CHUNK)], p1_v)
        pltpu.sync_copy(p2_hbm.at[pl.ds(cb, CHUNK)], p2_v)
        pltpu.sync_copy(x_hbm.at[pl.ds(cb, CHUNK)], xrows_v)
        c1 = pltpu.async_copy(xrows_v, xs_hbm.at[p1_v], sem)
        c2 = pltpu.async_copy(xrows_v, xs_hbm.at[p2_v], sem)
        c1.wait()
        c2.wait()


@jax.jit
def _scatter(x2d, p1, p2):
    return pl.kernel(
        _scatter_body,
        out_type=jax.ShapeDtypeStruct((PADTOT, D_MODEL), jnp.float32),
        mesh=plsc.VectorSubcoreMesh(core_axis_name="c", subcore_axis_name="s"),
        scratch_types=[
            pltpu.VMEM((CHUNK,), jnp.int32),
            pltpu.VMEM((CHUNK,), jnp.int32),
            pltpu.VMEM((CHUNK, D_MODEL), jnp.float32),
            pltpu.SemaphoreType.DMA,
        ],
    )(x2d, p1, p2)


def _expert_of(i, m_ref):
    ig = i * G
    e = jnp.int32(0)
    for k in range(N_EXP):
        e = e + jnp.where(ig >= m_ref[N_EXP + k], 1, 0).astype(jnp.int32)
    return jnp.minimum(e, N_EXP - 1)


def _ffn_kernel(m_ref, xs_ref, w1_ref, b1_ref, w2_ref, b2_ref, ys_ref):
    xb = xs_ref[...].astype(jnp.bfloat16)
    h = jnp.dot(xb, w1_ref[0], preferred_element_type=jnp.float32)
    h = _gelu_exact(h + b1_ref[0])
    y = jnp.dot(h.astype(jnp.bfloat16), w2_ref[0],
                preferred_element_type=jnp.float32)
    ys_ref[...] = y + b2_ref[0]


@jax.jit
def _ffn(ms16, xs, W1b, b1r, W2b, b2r):
    grid_spec = pltpu.PrefetchScalarGridSpec(
        num_scalar_prefetch=1,
        grid=(NTILES,),
        in_specs=[
            pl.BlockSpec((G, D_MODEL), lambda i, m: (i, 0)),
            pl.BlockSpec((1, D_MODEL, D_FF), lambda i, m: (_expert_of(i, m), 0, 0)),
            pl.BlockSpec((1, 1, D_FF), lambda i, m: (_expert_of(i, m), 0, 0)),
            pl.BlockSpec((1, D_FF, D_MODEL), lambda i, m: (_expert_of(i, m), 0, 0)),
            pl.BlockSpec((1, 1, D_MODEL), lambda i, m: (_expert_of(i, m), 0, 0)),
        ],
        out_specs=pl.BlockSpec((G, D_MODEL), lambda i, m: (i, 0)),
    )
    return pl.pallas_call(
        _ffn_kernel,
        grid_spec=grid_spec,
        out_shape=jax.ShapeDtypeStruct((PADTOT, D_MODEL), jnp.float32),
    )(ms16, xs, W1b, b1r, W2b, b2r)


def _gather_body(ys_hbm, p1_hbm, p2_hbm, y1_hbm, y2_hbm, p1_v, p2_v,
                 rows1_v, rows2_v, sem):
    wid = lax.axis_index("s") * 2 + lax.axis_index("c")
    base = wid * TPW
    for c in range(TPW // CHUNK):
        cb = base + c * CHUNK
        pltpu.sync_copy(p1_hbm.at[pl.ds(cb, CHUNK)], p1_v)
        pltpu.sync_copy(p2_hbm.at[pl.ds(cb, CHUNK)], p2_v)
        c1 = pltpu.async_copy(ys_hbm.at[p1_v], rows1_v, sem)
        c2 = pltpu.async_copy(ys_hbm.at[p2_v], rows2_v, sem)
        c1.wait()
        c2.wait()
        pltpu.sync_copy(rows1_v, y1_hbm.at[pl.ds(cb, CHUNK)])
        pltpu.sync_copy(rows2_v, y2_hbm.at[pl.ds(cb, CHUNK)])


@jax.jit
def _gather2(ys, p1, p2):
    return pl.kernel(
        _gather_body,
        out_type=[jax.ShapeDtypeStruct((TOK, D_MODEL), jnp.float32),
                  jax.ShapeDtypeStruct((TOK, D_MODEL), jnp.float32)],
        mesh=plsc.VectorSubcoreMesh(core_axis_name="c", subcore_axis_name="s"),
        scratch_types=[
            pltpu.VMEM((CHUNK,), jnp.int32),
            pltpu.VMEM((CHUNK,), jnp.int32),
            pltpu.VMEM((CHUNK, D_MODEL), jnp.float32),
            pltpu.VMEM((CHUNK, D_MODEL), jnp.float32),
            pltpu.SemaphoreType.DMA,
        ],
    )(ys, p1, p2)


def _combine_kernel(y1_ref, y2_ref, mf_ref, o_ref):
    lane = jax.lax.broadcasted_iota(jnp.int32, (BT, N_EXP), 1)
    w = mf_ref[...]
    w1 = jnp.sum(jnp.where(lane == 0, w, 0.0), axis=1, keepdims=True)
    w2 = jnp.sum(jnp.where(lane == 1, w, 0.0), axis=1, keepdims=True)
    o_ref[...] = w1 * y1_ref[...] + w2 * y2_ref[...]


@jax.jit
def _combine(y1, y2, mf):
    return pl.pallas_call(
        _combine_kernel,
        grid=(NT,),
        in_specs=[
            pl.BlockSpec((BT, D_MODEL), lambda t: (t, 0)),
            pl.BlockSpec((BT, D_MODEL), lambda t: (t, 0)),
            pl.BlockSpec((BT, N_EXP), lambda t: (t, 0)),
        ],
        out_specs=pl.BlockSpec((BT, D_MODEL), lambda t: (t, 0)),
        out_shape=jax.ShapeDtypeStruct((TOK, D_MODEL), jnp.float32),
    )(y1, y2, mf)


def kernel(x, Wg, bg, W1, b1, W2, b2):
    B, T, D = x.shape
    x2d = x.reshape(B * T, D)
    mi, mf, ms = _gate(x2d, Wg, bg.reshape(1, N_EXP))
    p = _positions(mi, ms)
    p1 = p[0]
    p2 = p[1]
    xs = _scatter(x2d, p1, p2)
    ys = _ffn(ms.reshape(16), xs,
              W1.astype(jnp.bfloat16), b1.reshape(N_EXP, 1, D_FF),
              W2.astype(jnp.bfloat16), b2.reshape(N_EXP, 1, D_MODEL))
    y1, y2 = _gather2(ys, p1, p2)
    out = _combine(y1, y2, mf)
    return out.reshape(B, T, D)


# speedup vs baseline: 1.0208x; 1.0208x over previous
import math

import jax
import jax.numpy as jnp
from jax import lax
from jax.experimental import pallas as pl
from jax.experimental.pallas import tpu as pltpu
from jax.experimental.pallas import tpu_sc as plsc

D_MODEL = 1024
D_FF = 4096
N_EXP = 8
TOK = 4096
BT = 256
NT = TOK // BT

G = 256
PADTOT = 2 * TOK + N_EXP * G
NTILES = PADTOT // G

NW = 32
TPW = TOK // NW
CHUNK = 32

_SQRT_HALF = 1.0 / math.sqrt(2.0)


def _gelu_exact(h):
    return 0.5 * h * (1.0 + jax.lax.erf(h * _SQRT_HALF))


def _first_occurrence_top2(s):
    jj = jax.lax.broadcasted_iota(jnp.int32, (N_EXP, N_EXP), 0)
    kk = jax.lax.broadcasted_iota(jnp.int32, (N_EXP, N_EXP), 1)
    tri = (jj < kk).astype(jnp.float32)
    neg = jnp.float32(-jnp.inf)

    m1 = jnp.max(s, axis=-1, keepdims=True)
    eq1 = (s == m1).astype(jnp.float32)
    prior1 = jnp.dot(eq1, tri, preferred_element_type=jnp.float32)
    first1 = eq1 * (prior1 == 0.0).astype(jnp.float32)

    s2 = jnp.where(first1 > 0.0, neg, s)
    m2 = jnp.max(s2, axis=-1, keepdims=True)
    eq2 = (s2 == m2).astype(jnp.float32)
    prior2 = jnp.dot(eq2, tri, preferred_element_type=jnp.float32)
    first2 = eq2 * (prior2 == 0.0).astype(jnp.float32)
    return first1, first2


def _gate_kernel(x_ref, wg_ref, bg_ref, mi_ref, mf_ref, ms_ref, carry_ref):
    t = pl.program_id(0)
    xb = x_ref[...]
    logits = jnp.dot(xb, wg_ref[...], preferred_element_type=jnp.float32)
    logits = logits + bg_ref[...]
    s = jax.nn.softmax(logits, axis=-1)
    first1, first2 = _first_occurrence_top2(s)

    lane = jax.lax.broadcasted_iota(jnp.int32, (BT, N_EXP), 1)
    lanef = lane.astype(jnp.float32)
    w1 = jnp.sum(s * first1, axis=-1, keepdims=True)
    w2 = jnp.sum(s * first2, axis=-1, keepdims=True)
    e1 = jnp.sum(lanef * first1, axis=-1, keepdims=True)
    e2 = jnp.sum(lanef * first2, axis=-1, keepdims=True)
    m = first1 + first2

    @pl.when(t == 0)
    def _():
        carry_ref[...] = jnp.zeros((1, N_EXP), jnp.float32)

    ii = jax.lax.broadcasted_iota(jnp.int32, (BT, BT), 0)
    jjb = jax.lax.broadcasted_iota(jnp.int32, (BT, BT), 1)
    trilS = (jjb < ii).astype(jnp.float32)
    carry = carry_ref[...]
    r = jnp.dot(trilS, m, preferred_element_type=jnp.float32) + carry
    carry_ref[...] = carry + jnp.sum(m, axis=0, keepdims=True)
    r1 = jnp.sum(r * first1, axis=-1, keepdims=True)
    r2 = jnp.sum(r * first2, axis=-1, keepdims=True)

    mf_ref[...] = jnp.where(lane == 0, w1, 0.0) + jnp.where(lane == 1, w2, 0.0)
    metai = (jnp.where(lane == 0, e1, 0.0) + jnp.where(lane == 1, e2, 0.0)
             + jnp.where(lane == 2, r1, 0.0) + jnp.where(lane == 3, r2, 0.0))

    ident = (ii == jjb).astype(jnp.float32)
    mi_t = lax.dot_general(metai, ident, (((0,), (0,)), ((), ())),
                           preferred_element_type=jnp.float32,
                           precision=lax.Precision.HIGHEST)
    mi_ref[...] = mi_t.astype(jnp.int32)

    @pl.when(t == NT - 1)
    def _():
        c = carry_ref[...]
        pc = jnp.floor((c + (G - 1)) / G) * G
        j8 = jax.lax.broadcasted_iota(jnp.int32, (N_EXP, N_EXP), 0)
        k8 = jax.lax.broadcasted_iota(jnp.int32, (N_EXP, N_EXP), 1)
        tri8 = (j8 < k8).astype(jnp.float32)
        off = jnp.dot(pc, tri8, preferred_element_type=jnp.float32)
        ends = off + pc
        j16 = jax.lax.broadcasted_iota(jnp.int32, (N_EXP, 16), 0)
        k16 = jax.lax.broadcasted_iota(jnp.int32, (N_EXP, 16), 1)
        p1 = (k16 == j16).astype(jnp.float32)
        p2 = (k16 == j16 + N_EXP).astype(jnp.float32)
        ms_ref[...] = (jnp.dot(off, p1, preferred_element_type=jnp.float32)
                       + jnp.dot(ends, p2, preferred_element_type=jnp.float32)
                       ).astype(jnp.int32)


@jax.jit
def _gate(x2d, Wg, bg2d):
    return pl.pallas_call(
        _gate_kernel,
        grid=(NT,),
        in_specs=[
            pl.BlockSpec((BT, D_MODEL), lambda t: (t, 0)),
            pl.BlockSpec((D_MODEL, N_EXP), lambda t: (0, 0)),
            pl.BlockSpec((1, N_EXP), lambda t: (0, 0)),
        ],
        out_specs=[
            pl.BlockSpec((N_EXP, BT), lambda t: (0, t)),
            pl.BlockSpec((BT, N_EXP), lambda t: (t, 0)),
            pl.BlockSpec((1, 16), lambda t: (0, 0)),
        ],
        out_shape=[
            jax.ShapeDtypeStruct((N_EXP, TOK), jnp.int32),
            jax.ShapeDtypeStruct((TOK, N_EXP), jnp.float32),
            jax.ShapeDtypeStruct((1, 16), jnp.int32),
        ],
        scratch_shapes=[pltpu.VMEM((1, N_EXP), jnp.float32)],
    )(x2d, Wg, bg2d)


def _pos_kernel(mi_ref, ms_ref, p_ref):
    e1 = mi_ref[0:1, :].astype(jnp.float32)
    e2 = mi_ref[1:2, :].astype(jnp.float32)
    r1 = mi_ref[2:3, :].astype(jnp.float32)
    r2 = mi_ref[3:4, :].astype(jnp.float32)
    offv = ms_ref[...].astype(jnp.float32)

    kk = jax.lax.broadcasted_iota(jnp.int32, (16, BT), 0).astype(jnp.float32)
    oh1 = (e1 == kk).astype(jnp.float32)
    oh2 = (e2 == kk).astype(jnp.float32)
    off1 = jnp.dot(offv, oh1, preferred_element_type=jnp.float32)
    off2 = jnp.dot(offv, oh2, preferred_element_type=jnp.float32)
    p1 = (r1 + off1).astype(jnp.int32)
    p2 = (r2 + off2).astype(jnp.int32)
    p_ref[...] = jnp.concatenate([p1, p2], axis=0)


@jax.jit
def _positions(mi, ms):
    return pl.pallas_call(
        _pos_kernel,
        grid=(NT,),
        in_specs=[
            pl.BlockSpec((N_EXP, BT), lambda t: (0, t)),
            pl.BlockSpec((1, 16), lambda t: (0, 0)),
        ],
        out_specs=pl.BlockSpec((2, BT), lambda t: (0, t)),
        out_shape=jax.ShapeDtypeStruct((2, TOK), jnp.int32),
    )(mi, ms)


def _scatter_body(x_hbm, p1_hbm, p2_hbm, xs_hbm, p1_v, p2_v, xrows_v, sem):
    wid = lax.axis_index("s") * 2 + lax.axis_index("c")
    base = wid * TPW
    for c in range(TPW // CHUNK):
        cb = base + c * CHUNK
        pltpu.sync_copy(p1_hbm.at[pl.ds(cb, CHUNK)], p1_v)
        pltpu.sync_copy(p2_hbm.at[pl.ds(cb, CHUNK)], p2_v)
        pltpu.sync_copy(x_hbm.at[pl.ds(cb, CHUNK)], xrows_v)
        c1 = pltpu.async_copy(xrows_v, xs_hbm.at[p1_v], sem)
        c2 = pltpu.async_copy(xrows_v, xs_hbm.at[p2_v], sem)
        c1.wait()
        c2.wait()


@jax.jit
def _scatter(x2d, p1, p2):
    return pl.kernel(
        _scatter_body,
        out_type=jax.ShapeDtypeStruct((PADTOT, D_MODEL), jnp.float32),
        mesh=plsc.VectorSubcoreMesh(core_axis_name="c", subcore_axis_name="s"),
        scratch_types=[
            pltpu.VMEM((CHUNK,), jnp.int32),
            pltpu.VMEM((CHUNK,), jnp.int32),
            pltpu.VMEM((CHUNK, D_MODEL), jnp.float32),
            pltpu.SemaphoreType.DMA,
        ],
    )(x2d, p1, p2)


def _expert_of(i, m_ref):
    ig = i * G
    e = jnp.int32(0)
    for k in range(N_EXP):
        e = e + jnp.where(ig >= m_ref[N_EXP + k], 1, 0).astype(jnp.int32)
    return jnp.minimum(e, N_EXP - 1)


def _ffn_kernel(m_ref, xs_ref, w1_ref, b1_ref, w2_ref, b2_ref, ys_ref):
    xb = xs_ref[...].astype(jnp.bfloat16)
    h = jnp.dot(xb, w1_ref[0], preferred_element_type=jnp.float32)
    h = _gelu_exact(h + b1_ref[0])
    y = jnp.dot(h.astype(jnp.bfloat16), w2_ref[0],
                preferred_element_type=jnp.float32)
    ys_ref[...] = y + b2_ref[0]


@jax.jit
def _ffn(ms16, xs, W1b, b1r, W2b, b2r):
    grid_spec = pltpu.PrefetchScalarGridSpec(
        num_scalar_prefetch=1,
        grid=(NTILES,),
        in_specs=[
            pl.BlockSpec((G, D_MODEL), lambda i, m: (i, 0)),
            pl.BlockSpec((1, D_MODEL, D_FF), lambda i, m: (_expert_of(i, m), 0, 0)),
            pl.BlockSpec((1, 1, D_FF), lambda i, m: (_expert_of(i, m), 0, 0)),
            pl.BlockSpec((1, D_FF, D_MODEL), lambda i, m: (_expert_of(i, m), 0, 0)),
            pl.BlockSpec((1, 1, D_MODEL), lambda i, m: (_expert_of(i, m), 0, 0)),
        ],
        out_specs=pl.BlockSpec((G, D_MODEL), lambda i, m: (i, 0)),
    )
    return pl.pallas_call(
        _ffn_kernel,
        grid_spec=grid_spec,
        out_shape=jax.ShapeDtypeStruct((PADTOT, D_MODEL), jnp.float32),
    )(ms16, xs, W1b, b1r, W2b, b2r)


def _gather_body(ys_hbm, p1_hbm, p2_hbm, y1_hbm, y2_hbm, p1_v, p2_v,
                 rows1_v, rows2_v, sem):
    wid = lax.axis_index("s") * 2 + lax.axis_index("c")
    base = wid * TPW
    for c in range(TPW // CHUNK):
        cb = base + c * CHUNK
        pltpu.sync_copy(p1_hbm.at[pl.ds(cb, CHUNK)], p1_v)
        pltpu.sync_copy(p2_hbm.at[pl.ds(cb, CHUNK)], p2_v)
        c1 = pltpu.async_copy(ys_hbm.at[p1_v], rows1_v, sem)
        c2 = pltpu.async_copy(ys_hbm.at[p2_v], rows2_v, sem)
        c1.wait()
        c2.wait()
        pltpu.sync_copy(rows1_v, y1_hbm.at[pl.ds(cb, CHUNK)])
        pltpu.sync_copy(rows2_v, y2_hbm.at[pl.ds(cb, CHUNK)])


@jax.jit
def _gather2(ys, p1, p2):
    return pl.kernel(
        _gather_body,
        out_type=[jax.ShapeDtypeStruct((TOK, D_MODEL), jnp.float32),
                  jax.ShapeDtypeStruct((TOK, D_MODEL), jnp.float32)],
        mesh=plsc.VectorSubcoreMesh(core_axis_name="c", subcore_axis_name="s"),
        scratch_types=[
            pltpu.VMEM((CHUNK,), jnp.int32),
            pltpu.VMEM((CHUNK,), jnp.int32),
            pltpu.VMEM((CHUNK, D_MODEL), jnp.float32),
            pltpu.VMEM((CHUNK, D_MODEL), jnp.float32),
            pltpu.SemaphoreType.DMA,
        ],
    )(ys, p1, p2)


def _combine_kernel(y1_ref, y2_ref, mf_ref, o_ref):
    lane = jax.lax.broadcasted_iota(jnp.int32, (BT, N_EXP), 1)
    w = mf_ref[...]
    w1 = jnp.sum(jnp.where(lane == 0, w, 0.0), axis=1, keepdims=True)
    w2 = jnp.sum(jnp.where(lane == 1, w, 0.0), axis=1, keepdims=True)
    o_ref[...] = w1 * y1_ref[...] + w2 * y2_ref[...]


@jax.jit
def _combine(y1, y2, mf):
    return pl.pallas_call(
        _combine_kernel,
        grid=(NT,),
        in_specs=[
            pl.BlockSpec((BT, D_MODEL), lambda t: (t, 0)),
            pl.BlockSpec((BT, D_MODEL), lambda t: (t, 0)),
            pl.BlockSpec((BT, N_EXP), lambda t: (t, 0)),
        ],
        out_specs=pl.BlockSpec((BT, D_MODEL), lambda t: (t, 0)),
        out_shape=jax.ShapeDtypeStruct((TOK, D_MODEL), jnp.float32),
    )(y1, y2, mf)


def kernel(x, Wg, bg, W1, b1, W2, b2):
    B, T, D = x.shape
    x2d = x.reshape(B * T, D)
    mi, mf, ms = _gate(x2d, Wg, bg.reshape(1, N_EXP))
    p = _positions(mi, ms)
    p1 = p[0]
    p2 = p[1]
    xs = _scatter(x2d, p1, p2)
    ys = _ffn(ms.reshape(16), xs,
              W1.astype(jnp.bfloat16), b1.reshape(N_EXP, 1, D_FF),
              W2.astype(jnp.bfloat16), b2.reshape(N_EXP, 1, D_MODEL))
    y1, y2 = _gather2(ys, p1, p2)
    out = _combine(y1, y2, mf)
    return out.reshape(B, T, D)
